# baseline (speedup 1.0000x reference)
import dataclasses
import functools

import jax
import jax.numpy as jnp
from jax import lax
from jax.experimental import pallas as pl
from jax.experimental.pallas import tpu as pltpu
from jax.experimental.pallas import tpu_sc as plsc

OUT_K = 256
IN_K = 1024
NLANES = 16

_MESH = plsc.VectorSubcoreMesh(core_axis_name="c", subcore_axis_name="s")
_CP = pltpu.CompilerParams()
if "needs_layout_passes" in pltpu.CompilerParams.__dataclass_fields__:
    _CP = dataclasses.replace(_CP, needs_layout_passes=False)


def _norm_key_body(w_ref, out_ref):
    w = w_ref[...]
    s = w * w
    acc = ((s[:, 0, :] + s[:, 4, :]) + (s[:, 2, :] + s[:, 6, :])) + (
        (s[:, 1, :] + s[:, 5, :]) + (s[:, 3, :] + s[:, 7, :]))
    norm = jnp.sqrt(acc)
    out_ref[...] = jnp.bitwise_not(lax.bitcast_convert_type(norm, jnp.int32))


def _norm_keys(wt, rows):
    block = 128
    grid = rows // block
    return pl.pallas_call(
        _norm_key_body,
        grid=(grid,),
        in_specs=[pl.BlockSpec((block, 8, IN_K), lambda i: (i, 0, 0))],
        out_specs=pl.BlockSpec((block, IN_K), lambda i: (i, 0)),
        out_shape=jax.ShapeDtypeStruct((rows, IN_K), jnp.int32),
    )(wt)


def _sc_topk_gather(keys, posf, wtsf, rows):
    rows_per = rows // 32

    @functools.partial(
        pl.kernel,
        out_type=(
            jax.ShapeDtypeStruct((rows * 3 * OUT_K,), jnp.float32),
            jax.ShapeDtypeStruct((rows * 8 * OUT_K,), jnp.float32),
        ),
        mesh=_MESH,
        compiler_params=_CP,
        scratch_types=[
            pltpu.VMEM((IN_K,), jnp.int32),
            pltpu.VMEM((IN_K,), jnp.int32),
            pltpu.VMEM((IN_K,), jnp.int32),
            pltpu.VMEM((IN_K,), jnp.int32),
            pltpu.VMEM((IN_K,), jnp.int32),
            pltpu.VMEM((8 * 256,), jnp.int32),
            pltpu.VMEM((256,), jnp.int32),
            pltpu.VMEM((256,), jnp.int32),
            pltpu.VMEM((256,), jnp.int32),
            pltpu.VMEM((256,), jnp.int32),
            pltpu.VMEM((256,), jnp.int32),
            pltpu.VMEM((256,), jnp.int32),
            pltpu.VMEM((256,), jnp.int32),
            pltpu.VMEM((256,), jnp.int32),
            pltpu.VMEM((IN_K,), jnp.int32),
            pltpu.VMEM((8 * IN_K,), jnp.float32),
            pltpu.VMEM((3 * IN_K,), jnp.float32),
            pltpu.VMEM((8 * OUT_K,), jnp.float32),
            pltpu.VMEM((3 * OUT_K,), jnp.float32),
            pltpu.SemaphoreType.DMA,
            pltpu.SemaphoreType.DMA,
            pltpu.SemaphoreType.DMA,
            pltpu.SemaphoreType.DMA,
        ],
    )
    def k(keys_hbm, pos_hbm, wts_hbm, outp_hbm, outw_hbm,
          kin, key_a, idx_a, key_b, idx_b, hist2d, offs_s0, offs_s1, offs_s2,
          offs_s3, offs_s4, offs_s5, offs_s6, offs_s7, cl_buf, wrow, prow,
          wout, pout, sem_k, sem_w, sem_p, sem_o):
        wid = lax.axis_index("c") * 16 + lax.axis_index("s")
        offs_s = (offs_s0, offs_s1, offs_s2, offs_s3,
                  offs_s4, offs_s5, offs_s6, offs_s7)
        row0 = wid * rows_per
        pltpu.async_copy(keys_hbm.at[row0], kin, sem_k)

        @pl.loop(0, rows_per)
        def _row(r):
            row = row0 + r
            pltpu.make_async_copy(keys_hbm.at[row], kin, sem_k).wait()
            cw = pltpu.async_copy(wts_hbm.at[pl.ds(row * (8 * IN_K), 8 * IN_K)],
                                  wrow, sem_w)
            cp = pltpu.async_copy(pos_hbm.at[pl.ds(row * (3 * IN_K), 3 * IN_K)],
                                  prow, sem_p)

            for p in range(4):
                if p == 0:
                    src_k, src_i = kin, idx_a
                    dst_k, dst_i = key_a, idx_a
                elif p == 1:
                    src_k, src_i = key_a, idx_a
                    dst_k, dst_i = key_b, idx_b
                elif p == 2:
                    src_k, src_i = key_b, idx_b
                    dst_k, dst_i = key_a, idx_a
                else:
                    src_k, src_i = key_a, idx_a
                    dst_k, dst_i = key_b, idx_b
                shift = 8 * p

                for j in range(128):
                    hist2d[pl.ds(16 * j, 16)] = jnp.zeros((16,), jnp.int32)

                @plsc.parallel_loop(0, 128, NLANES, unroll=2)
                def _hist(c0):
                    for u in range(8):
                        kk = src_k[pl.ds(c0 + u * 128, NLANES)]
                        d = (lax.shift_right_logical(kk, shift) & 255) + (
                            u * 256)
                        cnt, lastm = plsc.scan_count(d)
                        cnti = cnt.astype(jnp.int32)
                        cl_buf[pl.ds(c0 + u * 128, NLANES)] = cnti | (
                            lastm.astype(jnp.int32) << 31)
                        plsc.addupdate_scatter(hist2d, [d], cnti, mask=lastm)

                J = range(16)
                hs = [[hist2d[pl.ds(u * 256 + 16 * j, 16)] for u in range(8)]
                      for j in J]
                tots = [((hs[j][0] + hs[j][1]) + (hs[j][2] + hs[j][3])) +
                        ((hs[j][4] + hs[j][5]) + (hs[j][6] + hs[j][7]))
                        for j in J]
                incls = [plsc.cumsum(tots[j]) for j in J]
                carry = jnp.int32(0)
                for j in J:
                    ex = incls[j] - tots[j] + carry
                    offs_s0[pl.ds(16 * j, 16)] = ex
                    ex = ex + hs[j][0]
                    offs_s1[pl.ds(16 * j, 16)] = ex
                    ex = ex + hs[j][1]
                    offs_s2[pl.ds(16 * j, 16)] = ex
                    ex = ex + hs[j][2]
                    offs_s3[pl.ds(16 * j, 16)] = ex
                    ex = ex + hs[j][3]
                    offs_s4[pl.ds(16 * j, 16)] = ex
                    ex = ex + hs[j][4]
                    offs_s5[pl.ds(16 * j, 16)] = ex
                    ex = ex + hs[j][5]
                    offs_s6[pl.ds(16 * j, 16)] = ex
                    ex = ex + hs[j][6]
                    offs_s7[pl.ds(16 * j, 16)] = ex
                    carry = carry + incls[j][15]

                @pl.loop(0, 128, step=NLANES)
                def _perm(c0):
                    U = range(8)
                    kks = [src_k[pl.ds(c0 + u * 128, NLANES)] for u in U]
                    cls = [cl_buf[pl.ds(c0 + u * 128, NLANES)] for u in U]
                    if p == 0:
                        vvs = [lax.iota(jnp.int32, NLANES) + (c0 + u * 128)
                               for u in U]
                    else:
                        vvs = [src_i[pl.ds(c0 + u * 128, NLANES)] for u in U]
                    dds = [lax.shift_right_logical(kks[u], shift) & 255
                           for u in U]
                    bases = [plsc.load_gather(offs_s[u], [dds[u]]) for u in U]
                    cnts = [cls[u] & 0x7FFFFFFF for u in U]
                    lastms = [cls[u] < 0 for u in U]
                    poss = [bases[u] + cnts[u] - 1 for u in U]
                    for u in U:
                        plsc.store_scatter(dst_k, [poss[u]], kks[u])
                    for u in U:
                        plsc.store_scatter(dst_i, [poss[u]], vvs[u])
                    for u in U:
                        plsc.addupdate_scatter(offs_s[u], [dds[u]], cnts[u],
                                               mask=lastms[u])

                if p == 0:
                    @pl.when(r + 1 < rows_per)
                    def _():
                        pltpu.async_copy(keys_hbm.at[row + 1], kin, sem_k)

            cw.wait()
            cp.wait()

            @pl.when(r > 0)
            def _():
                pltpu.make_async_copy(
                    pout, outp_hbm.at[pl.ds(0, 3 * OUT_K)], sem_o).wait()
                pltpu.make_async_copy(
                    wout, outw_hbm.at[pl.ds(0, 8 * OUT_K)], sem_o).wait()

            @plsc.parallel_loop(0, OUT_K, NLANES, unroll=2)
            def _gather(c0):
                sel = idx_b[pl.ds(c0, NLANES)]
                for c in range(8):
                    wout[pl.ds(c * OUT_K + c0, NLANES)] = (
                        plsc.load_gather(wrow, [sel + c * IN_K]))
                for c in range(3):
                    pout[pl.ds(c * OUT_K + c0, NLANES)] = (
                        plsc.load_gather(prow, [sel + c * IN_K]))

            pltpu.async_copy(
                pout, outp_hbm.at[pl.ds(row * (3 * OUT_K), 3 * OUT_K)],
                sem_o)
            pltpu.async_copy(
                wout, outw_hbm.at[pl.ds(row * (8 * OUT_K), 8 * OUT_K)],
                sem_o)

        pltpu.make_async_copy(
            pout, outp_hbm.at[pl.ds(0, 3 * OUT_K)], sem_o).wait()
        pltpu.make_async_copy(
            wout, outw_hbm.at[pl.ds(0, 8 * OUT_K)], sem_o).wait()

    return k(keys, posf, wtsf)


def kernel(positions, weights):
    b, c, in_k, _ = positions.shape
    rows = b * c
    wt = weights.transpose(0, 1, 3, 2).reshape(rows, 8, in_k)
    posf = positions.transpose(0, 1, 3, 2).reshape(rows * 3 * in_k)
    wtsf = wt.reshape(rows * 8 * in_k)
    keys = _norm_keys(wt, rows)
    outp, outw = _sc_topk_gather(keys, posf, wtsf, rows)
    return (outp.reshape(b, c, 3, OUT_K).transpose(0, 1, 3, 2),
            outw.reshape(b, c, 8, OUT_K).transpose(0, 1, 3, 2))


# speedup vs baseline: 1.2112x; 1.2112x over previous
import dataclasses
import functools

import jax
import jax.numpy as jnp
from jax import lax
from jax.experimental import pallas as pl
from jax.experimental.pallas import tpu as pltpu
from jax.experimental.pallas import tpu_sc as plsc

OUT_K = 256
IN_K = 1024
NLANES = 16

_MESH = plsc.VectorSubcoreMesh(core_axis_name="c", subcore_axis_name="s")
_CP = pltpu.CompilerParams()
if "needs_layout_passes" in pltpu.CompilerParams.__dataclass_fields__:
    _CP = dataclasses.replace(_CP, needs_layout_passes=False)


def _norm_key_body(w_ref, out_ref):
    w = w_ref[...]
    s = w * w
    acc = ((s[:, 0, :] + s[:, 4, :]) + (s[:, 2, :] + s[:, 6, :])) + (
        (s[:, 1, :] + s[:, 5, :]) + (s[:, 3, :] + s[:, 7, :]))
    norm = jnp.sqrt(acc)
    out_ref[...] = jnp.bitwise_not(lax.bitcast_convert_type(norm, jnp.int32))


def _norm_keys(wt, rows):
    block = 128
    grid = rows // block
    return pl.pallas_call(
        _norm_key_body,
        grid=(grid,),
        in_specs=[pl.BlockSpec((block, 8, IN_K), lambda i: (i, 0, 0))],
        out_specs=pl.BlockSpec((block, IN_K), lambda i: (i, 0)),
        out_shape=jax.ShapeDtypeStruct((rows, IN_K), jnp.int32),
    )(wt)


def _sc_topk_gather(keys, posf, wtsf, rows):
    rows_per = rows // 32

    @functools.partial(
        pl.kernel,
        out_type=(
            jax.ShapeDtypeStruct((rows * 3 * OUT_K,), jnp.float32),
            jax.ShapeDtypeStruct((rows * 8 * OUT_K,), jnp.float32),
        ),
        mesh=_MESH,
        compiler_params=_CP,
        scratch_types=[
            pltpu.VMEM((IN_K,), jnp.int32),
            pltpu.VMEM((IN_K,), jnp.int32),
            pltpu.VMEM((IN_K,), jnp.int32),
            pltpu.VMEM((IN_K,), jnp.int32),
            pltpu.VMEM((IN_K,), jnp.int32),
            pltpu.VMEM((4 * 256,), jnp.int32),
            pltpu.VMEM((256,), jnp.int32),
            pltpu.VMEM((256,), jnp.int32),
            pltpu.VMEM((256,), jnp.int32),
            pltpu.VMEM((256,), jnp.int32),
            pltpu.VMEM((IN_K,), jnp.int32),
            pltpu.VMEM((8 * IN_K,), jnp.float32),
            pltpu.VMEM((3 * IN_K,), jnp.float32),
            pltpu.VMEM((8 * OUT_K,), jnp.float32),
            pltpu.VMEM((3 * OUT_K,), jnp.float32),
            pltpu.SemaphoreType.DMA,
            pltpu.SemaphoreType.DMA,
            pltpu.SemaphoreType.DMA,
            pltpu.SemaphoreType.DMA,
        ],
    )
    def k(keys_hbm, pos_hbm, wts_hbm, outp_hbm, outw_hbm,
          kin, key_a, idx_a, key_b, idx_b, hist2d, offs_s0, offs_s1, offs_s2,
          offs_s3, cl_buf, wrow, prow, wout, pout, sem_k, sem_w, sem_p,
          sem_o):
        wid = lax.axis_index("c") * 16 + lax.axis_index("s")
        offs_s = (offs_s0, offs_s1, offs_s2, offs_s3)
        row0 = wid * rows_per
        pltpu.async_copy(keys_hbm.at[row0], kin, sem_k)

        @pl.loop(0, rows_per)
        def _row(r):
            row = row0 + r
            pltpu.make_async_copy(keys_hbm.at[row], kin, sem_k).wait()
            cw = pltpu.async_copy(wts_hbm.at[pl.ds(row * (8 * IN_K), 8 * IN_K)],
                                  wrow, sem_w)
            cp = pltpu.async_copy(pos_hbm.at[pl.ds(row * (3 * IN_K), 3 * IN_K)],
                                  prow, sem_p)

            for p in range(4):
                if p == 0:
                    src_k, src_i = kin, idx_a
                    dst_k, dst_i = key_a, idx_a
                elif p == 1:
                    src_k, src_i = key_a, idx_a
                    dst_k, dst_i = key_b, idx_b
                elif p == 2:
                    src_k, src_i = key_b, idx_b
                    dst_k, dst_i = key_a, idx_a
                else:
                    src_k, src_i = key_a, idx_a
                    dst_k, dst_i = key_b, idx_b
                shift = 8 * p

                for j in range(64):
                    hist2d[pl.ds(16 * j, 16)] = jnp.zeros((16,), jnp.int32)

                @plsc.parallel_loop(0, 256, NLANES, unroll=2)
                def _hist(c0):
                    for u in range(4):
                        kk = src_k[pl.ds(c0 + u * 256, NLANES)]
                        d = (lax.shift_right_logical(kk, shift) & 255) + (
                            u * 256)
                        cnt, lastm = plsc.scan_count(d)
                        cnti = cnt.astype(jnp.int32)
                        cl_buf[pl.ds(c0 + u * 256, NLANES)] = cnti | (
                            lastm.astype(jnp.int32) << 31)
                        plsc.addupdate_scatter(hist2d, [d], cnti, mask=lastm)

                J = range(16)
                h0s = [hist2d[pl.ds(16 * j, 16)] for j in J]
                h1s = [hist2d[pl.ds(256 + 16 * j, 16)] for j in J]
                h2s = [hist2d[pl.ds(512 + 16 * j, 16)] for j in J]
                h3s = [hist2d[pl.ds(768 + 16 * j, 16)] for j in J]
                tots = [(h0s[j] + h1s[j]) + (h2s[j] + h3s[j]) for j in J]
                incls = [plsc.cumsum(tots[j]) for j in J]
                carry = jnp.int32(0)
                for j in J:
                    ex = incls[j] - tots[j] + carry
                    offs_s0[pl.ds(16 * j, 16)] = ex
                    ex1 = ex + h0s[j]
                    offs_s1[pl.ds(16 * j, 16)] = ex1
                    ex2 = ex1 + h1s[j]
                    offs_s2[pl.ds(16 * j, 16)] = ex2
                    offs_s3[pl.ds(16 * j, 16)] = ex2 + h2s[j]
                    carry = carry + incls[j][15]

                @pl.loop(0, 256, step=NLANES)
                def _perm(c0):
                    U = range(4)
                    kks = [src_k[pl.ds(c0 + u * 256, NLANES)] for u in U]
                    cls = [cl_buf[pl.ds(c0 + u * 256, NLANES)] for u in U]
                    if p == 0:
                        vvs = [lax.iota(jnp.int32, NLANES) + (c0 + u * 256)
                               for u in U]
                    else:
                        vvs = [src_i[pl.ds(c0 + u * 256, NLANES)] for u in U]
                    dds = [lax.shift_right_logical(kks[u], shift) & 255
                           for u in U]
                    bases = [plsc.load_gather(offs_s[u], [dds[u]]) for u in U]
                    cnts = [cls[u] & 0x7FFFFFFF for u in U]
                    lastms = [cls[u] < 0 for u in U]
                    poss = [bases[u] + cnts[u] - 1 for u in U]
                    for u in U:
                        plsc.store_scatter(dst_k, [poss[u]], kks[u])
                    for u in U:
                        plsc.store_scatter(dst_i, [poss[u]], vvs[u])
                    for u in U:
                        plsc.addupdate_scatter(offs_s[u], [dds[u]], cnts[u],
                                               mask=lastms[u])

                if p == 0:
                    @pl.when(r + 1 < rows_per)
                    def _():
                        pltpu.async_copy(keys_hbm.at[row + 1], kin, sem_k)

            cw.wait()
            cp.wait()

            @pl.when(r > 0)
            def _():
                pltpu.make_async_copy(
                    pout, outp_hbm.at[pl.ds(0, 3 * OUT_K)], sem_o).wait()
                pltpu.make_async_copy(
                    wout, outw_hbm.at[pl.ds(0, 8 * OUT_K)], sem_o).wait()

            @plsc.parallel_loop(0, OUT_K, NLANES, unroll=2)
            def _gather(c0):
                sel = idx_b[pl.ds(c0, NLANES)]
                for c in range(8):
                    wout[pl.ds(c * OUT_K + c0, NLANES)] = (
                        plsc.load_gather(wrow, [sel + c * IN_K]))
                for c in range(3):
                    pout[pl.ds(c * OUT_K + c0, NLANES)] = (
                        plsc.load_gather(prow, [sel + c * IN_K]))

            pltpu.async_copy(
                pout, outp_hbm.at[pl.ds(row * (3 * OUT_K), 3 * OUT_K)],
                sem_o)
            pltpu.async_copy(
                wout, outw_hbm.at[pl.ds(row * (8 * OUT_K), 8 * OUT_K)],
                sem_o)

        pltpu.make_async_copy(
            pout, outp_hbm.at[pl.ds(0, 3 * OUT_K)], sem_o).wait()
        pltpu.make_async_copy(
            wout, outw_hbm.at[pl.ds(0, 8 * OUT_K)], sem_o).wait()

    return k(keys, posf, wtsf)


def kernel(positions, weights):
    b, c, in_k, _ = positions.shape
    rows = b * c
    wt = weights.transpose(0, 1, 3, 2).reshape(rows, 8, in_k)
    posf = positions.transpose(0, 1, 3, 2).reshape(rows * 3 * in_k)
    wtsf = wt.reshape(rows * 8 * in_k)
    keys = _norm_keys(wt, rows)
    outp, outw = _sc_topk_gather(keys, posf, wtsf, rows)
    return (outp.reshape(b, c, 3, OUT_K).transpose(0, 1, 3, 2),
            outw.reshape(b, c, 8, OUT_K).transpose(0, 1, 3, 2))
